# baseline (speedup 1.0000x reference)
import functools

import jax
import jax.numpy as jnp
from jax import lax
from jax.experimental import pallas as pl
from jax.experimental.pallas import tpu as pltpu
from jax.experimental.pallas import tpu_sc as plsc

NC = 2
NS = 16
NW = NC * NS
CH = 128
LANES = 16


def _cdiv(a, b):
    return (a + b - 1) // b


def _sc_degree(dst2d, n_pad, k0, k1):
    mesh = plsc.VectorSubcoreMesh(core_axis_name="c", subcore_axis_name="s")
    zslice = n_pad // NS
    kmax = max(k0, k1)

    @functools.partial(
        pl.kernel,
        out_type=jax.ShapeDtypeStruct((NC * n_pad,), jnp.float32),
        mesh=mesh,
        scratch_types=[
            pltpu.VMEM((kmax, CH), jnp.int32),
            pltpu.VMEM((CH,), jnp.float32),
            pltpu.VMEM((_cdiv(zslice, LANES) * LANES,), jnp.float32),
            pltpu.VMEM_SHARED((n_pad,), jnp.float32),
            pltpu.SemaphoreType.DMA,
        ],
        compiler_params=pltpu.CompilerParams(use_tc_tiling_on_sc=False),
    )
    def deg_kernel(dst_hbm, out_hbm, didx, ones, zbuf, acc, sem):
        c = lax.axis_index("c")
        s = lax.axis_index("s")
        kpt = jnp.where(c == 0, k0, k1)
        base = jnp.where(c == 0, s * k0, NS * k0 + s * k1)

        def fill_ones(i, carry):
            ones[pl.ds(i * LANES, LANES)] = jnp.full((LANES,), 1.0, jnp.float32)
            return carry

        lax.fori_loop(0, CH // LANES, fill_ones, 0)

        def fill_zero(i, carry):
            zbuf[pl.ds(i * LANES, LANES)] = jnp.zeros((LANES,), jnp.float32)
            return carry

        lax.fori_loop(0, _cdiv(zslice, LANES), fill_zero, 0)
        pltpu.sync_copy(zbuf.at[pl.ds(0, zslice)],
                        acc.at[pl.ds(s * zslice, zslice)])
        pltpu.sync_copy(dst_hbm.at[pl.ds(base, kmax)], didx)
        plsc.subcore_barrier()

        BATCH = 8

        def batch(q, carry):
            for b in range(BATCH):
                k = q * BATCH + b

                @pl.when(k < kpt)
                def _():
                    pltpu.async_copy(ones, acc.at[didx.at[k]], sem, add=True)
            for b in range(BATCH):
                k = q * BATCH + b

                @pl.when(k < kpt)
                def _():
                    pltpu.make_async_copy(ones, acc.at[didx.at[k]],
                                          sem).wait()
            return carry

        lax.fori_loop(0, (kpt + BATCH - 1) // BATCH, batch, 0)
        plsc.subcore_barrier()
        pltpu.sync_copy(acc.at[pl.ds(s * zslice, zslice)],
                        out_hbm.at[pl.ds(c * n_pad + s * zslice, zslice)])

    return deg_kernel(dst2d)


def _sc_aggregate(src2d, dst2d, y, n, n_pad, k0, k1, ch, nbuf):
    d = y.shape[1]
    mesh = plsc.VectorSubcoreMesh(core_axis_name="c", subcore_axis_name="s")
    zslice = n_pad // NS
    kmax = max(k0, k1)

    @functools.partial(
        pl.kernel,
        out_type=jax.ShapeDtypeStruct((NC * n_pad, d), jnp.float32),
        mesh=mesh,
        scratch_types=[
            [pltpu.VMEM((ch,), jnp.int32)] * nbuf,
            pltpu.VMEM((kmax, ch), jnp.int32),
            [pltpu.VMEM((ch, d), jnp.float32)] * nbuf,
            pltpu.VMEM_SHARED((n_pad, d), jnp.float32),
            [pltpu.SemaphoreType.DMA] * nbuf,
            [pltpu.SemaphoreType.DMA] * nbuf,
            [pltpu.SemaphoreType.DMA] * nbuf,
        ],
        compiler_params=pltpu.CompilerParams(use_tc_tiling_on_sc=False),
    )
    def agg_kernel(src_hbm, dst_hbm, y_hbm, out_hbm, sidx, didx, rows, acc,
                   gsem, ssem, isem):
        c = lax.axis_index("c")
        s = lax.axis_index("s")
        kpt = jnp.where(c == 0, k0, k1)
        base = jnp.where(c == 0, s * k0, NS * k0 + s * k1)

        def fill_zero(i, carry):
            j = i // (d // LANES)
            l = i % (d // LANES)
            rows[0][j, pl.ds(l * LANES, LANES)] = jnp.zeros((LANES,),
                                                            jnp.float32)
            return carry

        lax.fori_loop(0, ch * d // LANES, fill_zero, 0)

        def zero_acc(j, carry):
            pltpu.sync_copy(rows[0], acc.at[pl.ds(s * zslice + j * ch, ch)])
            return carry

        lax.fori_loop(0, zslice // ch, zero_acc, 0)
        rem = zslice % ch
        if rem:
            pltpu.sync_copy(
                rows[0].at[pl.ds(0, rem)],
                acc.at[pl.ds(s * zslice + (zslice // ch) * ch, rem)])
        pltpu.sync_copy(dst_hbm.at[pl.ds(base, kmax)], didx)
        plsc.subcore_barrier()

        def gather(b):
            pltpu.async_copy(y_hbm.at[sidx[b]], rows[b], gsem[b])

        def gather_wait(b):
            pltpu.make_async_copy(y_hbm.at[sidx[b]], rows[b], gsem[b]).wait()

        def scatter(k, b):
            pltpu.async_copy(rows[b], acc.at[didx.at[k]], ssem[b], add=True)

        def scatter_wait(k, b):
            pltpu.make_async_copy(rows[b], acc.at[didx.at[k]],
                                  ssem[b]).wait()

        def sidx_copy(k, b):
            pltpu.async_copy(src_hbm.at[base + k], sidx[b], isem[b])

        def sidx_wait(k, b):
            pltpu.make_async_copy(src_hbm.at[base + k], sidx[b],
                                  isem[b]).wait()

        for b in range(nbuf):
            @pl.when(b < kpt)
            def _():
                sidx_copy(b, b)
                sidx_wait(b, b)
                gather(b)

        def ring(q, carry):
            for b in range(nbuf):
                k = q * nbuf + b

                @pl.when(k < kpt)
                def _():
                    gather_wait(b)
                    scatter(k, b)

                    @pl.when(k + nbuf < kpt)
                    def _():
                        sidx_copy(k + nbuf, b)
            for b in range(nbuf):
                k = q * nbuf + b

                @pl.when(k < kpt)
                def _():
                    scatter_wait(k, b)

                    @pl.when(k + nbuf < kpt)
                    def _():
                        sidx_wait(k + nbuf, b)
                        gather(b)
            return carry

        lax.fori_loop(0, (kpt + nbuf - 1) // nbuf, ring, 0)
        plsc.subcore_barrier()
        pltpu.sync_copy(acc.at[pl.ds(s * zslice, zslice)],
                        out_hbm.at[pl.ds(c * n_pad + s * zslice, zslice)])

    return agg_kernel(src2d, dst2d, y)


def _tc_first(x, w, deg_a, deg_b):
    n = x.shape[0]

    def body(x_ref, w_ref, da_ref, db_ref, y_ref, v_ref):
        dinv = lax.rsqrt(da_ref[...] + db_ref[...] + 1.0)
        v_ref[...] = dinv
        y_ref[...] = jnp.dot(x_ref[...], w_ref[...],
                             preferred_element_type=jnp.float32) * dinv

    return pl.pallas_call(
        body,
        out_shape=[jax.ShapeDtypeStruct((n, w.shape[1]), jnp.float32),
                   jax.ShapeDtypeStruct((n, 1), jnp.float32)],
    )(x, w, deg_a, deg_b)


def _tc_mid(agg, y1, dinv_col, b1row, w2, n, n_pad):
    def body(agg_ref, y_ref, v_ref, bias_ref, w_ref, o_ref):
        acc = agg_ref[pl.ds(0, n), :] + agg_ref[pl.ds(n_pad, n), :]
        h = (acc + y_ref[...]) * v_ref[...] + bias_ref[...]
        h = jnp.maximum(h, 0.0)
        o_ref[...] = jnp.dot(h, w_ref[...],
                             preferred_element_type=jnp.float32) * v_ref[...]

    return pl.pallas_call(
        body,
        out_shape=jax.ShapeDtypeStruct((n, w2.shape[1]), jnp.float32),
    )(agg, y1, dinv_col, b1row, w2)


def _tc_final(agg, y2, dinv_col, b2row, n, n_pad):
    def body(agg_ref, y_ref, v_ref, bias_ref, o_ref):
        acc = agg_ref[pl.ds(0, n), :] + agg_ref[pl.ds(n_pad, n), :]
        o_ref[...] = (acc + y_ref[...]) * v_ref[...] + bias_ref[...]

    return pl.pallas_call(
        body,
        out_shape=jax.ShapeDtypeStruct(y2.shape, jnp.float32),
    )(agg, y2, dinv_col, b2row)


def kernel(x, edge_index, W1, b1, W2, b2):
    n = x.shape[0]
    e = edge_index.shape[1]

    src = edge_index[0].astype(jnp.int32)
    dst = edge_index[1].astype(jnp.int32)

    def chunked(arr, ch, fill, f0):
        n_chunks = _cdiv(e, ch)
        c0 = int(n_chunks * f0)
        k0 = max(_cdiv(c0, NS), 1)
        k1 = max(_cdiv(n_chunks - NS * k0, NS), 1)
        rows = NS * (k0 + k1) + abs(k0 - k1)
        padn = rows * ch - e
        return (jnp.concatenate(
            [arr, jnp.full((padn,), fill, jnp.int32)]).reshape(rows, ch),
            k0, k1)

    F0_A = 0.6
    F0_B = 0.65
    srcA, kA0, kA1 = chunked(src, 64, 0, F0_A)
    dstA, _, _ = chunked(dst, 64, n, F0_A)
    srcB, kB0, kB1 = chunked(src, 128, 0, F0_B)
    dstB, _, _ = chunked(dst, 128, n, F0_B)

    n_pad = _cdiv(n + 1, NS * 8) * NS * 8

    deg_parts = _sc_degree(dstB, n_pad, kB0, kB1)
    deg_a = deg_parts[:n].reshape(n, 1)
    deg_b = deg_parts[n_pad:n_pad + n].reshape(n, 1)

    y1, dinv_col = _tc_first(x, W1, deg_a, deg_b)
    agg1 = _sc_aggregate(srcA, dstA, y1, n, n_pad, kA0, kA1, 64, 4)
    y2 = _tc_mid(agg1, y1, dinv_col, b1.reshape(1, -1), W2, n, n_pad)
    agg2 = _sc_aggregate(srcB, dstB, y2, n, n_pad, kB0, kB1, 128, 8)
    out = _tc_final(agg2, y2, dinv_col, b2.reshape(1, -1), n, n_pad)
    return out


# speedup vs baseline: 1.0037x; 1.0037x over previous
import functools

import jax
import jax.numpy as jnp
from jax import lax
from jax.experimental import pallas as pl
from jax.experimental.pallas import tpu as pltpu
from jax.experimental.pallas import tpu_sc as plsc

NC = 2
NS = 16
NW = NC * NS
CH = 128
LANES = 16


def _cdiv(a, b):
    return (a + b - 1) // b


def _sc_degree(dst2d, n_pad, k0, k1):
    mesh = plsc.VectorSubcoreMesh(core_axis_name="c", subcore_axis_name="s")
    zslice = n_pad // NS
    kmax = max(k0, k1)

    @functools.partial(
        pl.kernel,
        out_type=jax.ShapeDtypeStruct((NC * n_pad,), jnp.float32),
        mesh=mesh,
        scratch_types=[
            pltpu.VMEM((kmax, CH), jnp.int32),
            pltpu.VMEM((CH,), jnp.float32),
            pltpu.VMEM((_cdiv(zslice, LANES) * LANES,), jnp.float32),
            pltpu.VMEM_SHARED((n_pad,), jnp.float32),
            pltpu.SemaphoreType.DMA,
        ],
        compiler_params=pltpu.CompilerParams(use_tc_tiling_on_sc=False),
    )
    def deg_kernel(dst_hbm, out_hbm, didx, ones, zbuf, acc, sem):
        c = lax.axis_index("c")
        s = lax.axis_index("s")
        kpt = jnp.where(c == 0, k0, k1)
        base = jnp.where(c == 0, s * k0, NS * k0 + s * k1)

        def fill_ones(i, carry):
            ones[pl.ds(i * LANES, LANES)] = jnp.full((LANES,), 1.0, jnp.float32)
            return carry

        lax.fori_loop(0, CH // LANES, fill_ones, 0)

        def fill_zero(i, carry):
            zbuf[pl.ds(i * LANES, LANES)] = jnp.zeros((LANES,), jnp.float32)
            return carry

        lax.fori_loop(0, _cdiv(zslice, LANES), fill_zero, 0)
        pltpu.sync_copy(zbuf.at[pl.ds(0, zslice)],
                        acc.at[pl.ds(s * zslice, zslice)])
        pltpu.sync_copy(dst_hbm.at[pl.ds(base, kmax)], didx)
        plsc.subcore_barrier()

        BATCH = 8

        def batch(q, carry):
            for b in range(BATCH):
                k = q * BATCH + b

                @pl.when(k < kpt)
                def _():
                    pltpu.async_copy(ones, acc.at[didx.at[k]], sem, add=True)
            for b in range(BATCH):
                k = q * BATCH + b

                @pl.when(k < kpt)
                def _():
                    pltpu.make_async_copy(ones, acc.at[didx.at[k]],
                                          sem).wait()
            return carry

        lax.fori_loop(0, (kpt + BATCH - 1) // BATCH, batch, 0)
        plsc.subcore_barrier()
        pltpu.sync_copy(acc.at[pl.ds(s * zslice, zslice)],
                        out_hbm.at[pl.ds(c * n_pad + s * zslice, zslice)])

    return deg_kernel(dst2d)


def _sc_aggregate(src2d, dst2d, y, n, n_pad, k0, k1, ch, nbuf):
    d = y.shape[1]
    mesh = plsc.VectorSubcoreMesh(core_axis_name="c", subcore_axis_name="s")
    zslice = n_pad // NS
    kmax = max(k0, k1)

    @functools.partial(
        pl.kernel,
        out_type=jax.ShapeDtypeStruct((NC * n_pad, d), jnp.float32),
        mesh=mesh,
        scratch_types=[
            [pltpu.VMEM((ch,), jnp.int32)] * nbuf,
            pltpu.VMEM((kmax, ch), jnp.int32),
            [pltpu.VMEM((ch, d), jnp.float32)] * nbuf,
            pltpu.VMEM_SHARED((n_pad, d), jnp.float32),
            [pltpu.SemaphoreType.DMA] * nbuf,
            [pltpu.SemaphoreType.DMA] * nbuf,
            [pltpu.SemaphoreType.DMA] * nbuf,
        ],
        compiler_params=pltpu.CompilerParams(use_tc_tiling_on_sc=False),
    )
    def agg_kernel(src_hbm, dst_hbm, y_hbm, out_hbm, sidx, didx, rows, acc,
                   gsem, ssem, isem):
        c = lax.axis_index("c")
        s = lax.axis_index("s")
        kpt = jnp.where(c == 0, k0, k1)
        base = jnp.where(c == 0, s * k0, NS * k0 + s * k1)

        def fill_zero(i, carry):
            j = i // (d // LANES)
            l = i % (d // LANES)
            rows[0][j, pl.ds(l * LANES, LANES)] = jnp.zeros((LANES,),
                                                            jnp.float32)
            return carry

        lax.fori_loop(0, ch * d // LANES, fill_zero, 0)

        def zero_acc(j, carry):
            pltpu.sync_copy(rows[0], acc.at[pl.ds(s * zslice + j * ch, ch)])
            return carry

        lax.fori_loop(0, zslice // ch, zero_acc, 0)
        rem = zslice % ch
        if rem:
            pltpu.sync_copy(
                rows[0].at[pl.ds(0, rem)],
                acc.at[pl.ds(s * zslice + (zslice // ch) * ch, rem)])
        pltpu.sync_copy(dst_hbm.at[pl.ds(base, kmax)], didx)
        plsc.subcore_barrier()

        def gather(b):
            pltpu.async_copy(y_hbm.at[sidx[b]], rows[b], gsem[b])

        def gather_wait(b):
            pltpu.make_async_copy(y_hbm.at[sidx[b]], rows[b], gsem[b]).wait()

        def scatter(k, b):
            pltpu.async_copy(rows[b], acc.at[didx.at[k]], ssem[b], add=True)

        def scatter_wait(k, b):
            pltpu.make_async_copy(rows[b], acc.at[didx.at[k]],
                                  ssem[b]).wait()

        def sidx_copy(k, b):
            pltpu.async_copy(src_hbm.at[base + k], sidx[b], isem[b])

        def sidx_wait(k, b):
            pltpu.make_async_copy(src_hbm.at[base + k], sidx[b],
                                  isem[b]).wait()

        for b in range(nbuf):
            @pl.when(b < kpt)
            def _():
                sidx_copy(b, b)
                sidx_wait(b, b)
                gather(b)

        def ring(q, carry):
            for b in range(nbuf):
                k = q * nbuf + b

                @pl.when(k < kpt)
                def _():
                    gather_wait(b)
                    scatter(k, b)

                    @pl.when(k + nbuf < kpt)
                    def _():
                        sidx_copy(k + nbuf, b)
            for b in range(nbuf):
                k = q * nbuf + b

                @pl.when(k < kpt)
                def _():
                    scatter_wait(k, b)

                    @pl.when(k + nbuf < kpt)
                    def _():
                        sidx_wait(k + nbuf, b)
                        gather(b)
            return carry

        lax.fori_loop(0, (kpt + nbuf - 1) // nbuf, ring, 0)
        plsc.subcore_barrier()
        pltpu.sync_copy(acc.at[pl.ds(s * zslice, zslice)],
                        out_hbm.at[pl.ds(c * n_pad + s * zslice, zslice)])

    return agg_kernel(src2d, dst2d, y)


def _tc_first(x, w, deg_a, deg_b):
    n = x.shape[0]

    def body(x_ref, w_ref, da_ref, db_ref, y_ref, v_ref):
        dinv = lax.rsqrt(da_ref[...] + db_ref[...] + 1.0)
        v_ref[...] = dinv
        y_ref[...] = jnp.dot(x_ref[...], w_ref[...],
                             preferred_element_type=jnp.float32) * dinv

    return pl.pallas_call(
        body,
        out_shape=[jax.ShapeDtypeStruct((n, w.shape[1]), jnp.float32),
                   jax.ShapeDtypeStruct((n, 1), jnp.float32)],
    )(x, w, deg_a, deg_b)


def _tc_mid(agg, y1, dinv_col, b1row, w2, n, n_pad):
    def body(agg_ref, y_ref, v_ref, bias_ref, w_ref, o_ref):
        acc = agg_ref[pl.ds(0, n), :] + agg_ref[pl.ds(n_pad, n), :]
        h = (acc + y_ref[...]) * v_ref[...] + bias_ref[...]
        h = jnp.maximum(h, 0.0)
        o_ref[...] = jnp.dot(h, w_ref[...],
                             preferred_element_type=jnp.float32) * v_ref[...]

    return pl.pallas_call(
        body,
        out_shape=jax.ShapeDtypeStruct((n, w2.shape[1]), jnp.float32),
    )(agg, y1, dinv_col, b1row, w2)


def _tc_final(agg, y2, dinv_col, b2row, n, n_pad):
    def body(agg_ref, y_ref, v_ref, bias_ref, o_ref):
        acc = agg_ref[pl.ds(0, n), :] + agg_ref[pl.ds(n_pad, n), :]
        o_ref[...] = (acc + y_ref[...]) * v_ref[...] + bias_ref[...]

    return pl.pallas_call(
        body,
        out_shape=jax.ShapeDtypeStruct(y2.shape, jnp.float32),
    )(agg, y2, dinv_col, b2row)


def kernel(x, edge_index, W1, b1, W2, b2):
    n = x.shape[0]
    e = edge_index.shape[1]

    src = edge_index[0].astype(jnp.int32)
    dst = edge_index[1].astype(jnp.int32)

    def chunked(arr, ch, fill, f0):
        n_chunks = _cdiv(e, ch)
        c0 = int(n_chunks * f0)
        k0 = max(_cdiv(c0, NS), 1)
        k1 = max(_cdiv(n_chunks - NS * k0, NS), 1)
        rows = NS * (k0 + k1) + abs(k0 - k1)
        padn = rows * ch - e
        return (jnp.concatenate(
            [arr, jnp.full((padn,), fill, jnp.int32)]).reshape(rows, ch),
            k0, k1)

    F0_A = 0.6
    F0_B = 0.65
    srcA, kA0, kA1 = chunked(src, 64, 0, F0_A)
    dstA, _, _ = chunked(dst, 64, n, F0_A)
    srcB, kB0, kB1 = chunked(src, 128, 0, F0_B)
    dstB, _, _ = chunked(dst, 128, n, F0_B)

    n_pad = _cdiv(n + 1, NS * 8) * NS * 8

    deg_parts = _sc_degree(dstB, n_pad, kB0, kB1)
    deg_a = deg_parts[:n].reshape(n, 1)
    deg_b = deg_parts[n_pad:n_pad + n].reshape(n, 1)

    y1, dinv_col = _tc_first(x, W1, deg_a, deg_b)
    agg1 = _sc_aggregate(srcA, dstA, y1, n, n_pad, kA0, kA1, 64, 4)
    y2 = _tc_mid(agg1, y1, dinv_col, b1.reshape(1, -1), W2, n, n_pad)
    agg2 = _sc_aggregate(srcB, dstB, y2, n, n_pad, kB0, kB1, 128, 6)
    out = _tc_final(agg2, y2, dinv_col, b2.reshape(1, -1), n, n_pad)
    return out
